# baseline (speedup 1.0000x reference)
import functools

import jax
import jax.numpy as jnp
import numpy as np
from jax.experimental import pallas as pl
from jax.experimental.pallas import tpu as pltpu

_INTERPRET = False

WIDE = 144
_BE = 1024


def _silu(x):
    return x * jax.nn.sigmoid(x)


def _edge_body(g1_ref, g2_ref, vecs_ref, w1_ref, wc0_ref, out_ref):
    g = g1_ref[...] + g2_ref[...]
    vecs = vecs_ref[...]
    cd = g[:, 128:131]
    radial = jnp.sum(cd * cd, axis=1, keepdims=True)
    pre = g[:, :128] + vecs[0] + radial * vecs[1]
    x1 = _silu(pre)
    x2 = _silu(jnp.dot(x1, w1_ref[...], preferred_element_type=jnp.float32)
               + vecs[2])
    att_pre = jnp.sum(x2 * vecs[3], axis=1, keepdims=True) + vecs_ref[6, 0]
    ef = x2 * jax.nn.sigmoid(att_pre)
    cm = _silu(jnp.dot(ef, wc0_ref[...], preferred_element_type=jnp.float32)
               + vecs[4])
    s = jnp.tanh(jnp.sum(cm * vecs[5], axis=1, keepdims=True))
    be = g.shape[0]
    out_ref[...] = jnp.concatenate(
        [ef, cd * s, jnp.ones((be, 1), jnp.float32),
         jnp.zeros((be, WIDE - 132), jnp.float32)], axis=1)


def _edge_mlp(G1, G2, p):
    epad = G1.shape[0]
    nf = 128
    W0 = p["edge_mlp0"]["W"]
    vecs = jnp.zeros((8, 128), jnp.float32)
    vecs = vecs.at[0].set(p["edge_mlp0"]["b"])
    vecs = vecs.at[1].set(W0[2 * nf])
    vecs = vecs.at[2].set(p["edge_mlp1"]["b"])
    vecs = vecs.at[3].set(p["att_mlp"]["W"][:, 0])
    vecs = vecs.at[4].set(p["coord_mlp0"]["b"])
    vecs = vecs.at[5].set(p["coord_mlp1"]["W"][:, 0])
    vecs = vecs.at[6, 0].set(p["att_mlp"]["b"][0])
    be = min(_BE, epad)
    grid = epad // be
    return pl.pallas_call(
        _edge_body,
        grid=(grid,),
        in_specs=[
            pl.BlockSpec((be, WIDE), lambda i: (i, 0)),
            pl.BlockSpec((be, WIDE), lambda i: (i, 0)),
            pl.BlockSpec((8, 128), lambda i: (0, 0)),
            pl.BlockSpec((128, 128), lambda i: (0, 0)),
            pl.BlockSpec((128, 128), lambda i: (0, 0)),
        ],
        out_specs=pl.BlockSpec((be, WIDE), lambda i: (i, 0)),
        out_shape=jax.ShapeDtypeStruct((epad, WIDE), jnp.float32),
        interpret=_INTERPRET,
    )(G1, G2, vecs, p["edge_mlp1"]["W"], p["coord_mlp0"]["W"])


def _lin(p, x):
    y = x @ p["W"]
    if "b" in p:
        y = y + p["b"]
    return y


def _seg_sum(data, ids, n):
    return jax.ops.segment_sum(data, ids, num_segments=n)


def _seg_mean(data, ids, n):
    s = jax.ops.segment_sum(data, ids, num_segments=n)
    c = jax.ops.segment_sum(jnp.ones((data.shape[0], 1), data.dtype), ids,
                            num_segments=n)
    return s / jnp.clip(c, 1.0, None)


def _bn(p, x):
    m = jnp.mean(x, axis=0)
    v = jnp.var(x, axis=0)
    return p["gamma"] * (x - m) * jax.lax.rsqrt(v + 1e-5) + p["beta"]


def _egcl_big(p, h, coord, rowg, colg, rowscat, nacc):
    n, nf = h.shape
    W0 = p["edge_mlp0"]["W"]
    zpad = jnp.zeros((n, WIDE - nf - 3), jnp.float32)
    Ahat = jnp.concatenate([h @ W0[:nf], coord, zpad], axis=1)
    Bhat = jnp.concatenate([h @ W0[nf:2 * nf], -coord, zpad], axis=1)
    G1 = jnp.take(Ahat, rowg, axis=0)
    G2 = jnp.take(Bhat, colg, axis=0)
    out = _edge_mlp(G1, G2, p)
    acc = _seg_sum(out, rowscat, nacc)[:n]
    agg = acc[:, :nf]
    cd_sum = acc[:, nf:nf + 3]
    cnt = acc[:, nf + 3:nf + 4]
    coord = coord + cd_sum / jnp.clip(cnt, 1.0, None)
    o = _silu(_lin(p["node_mlp0"], jnp.concatenate([h, agg], axis=1)))
    h = h + _lin(p["node_mlp1"], o)
    return h, coord


def _egnn_big(p, x, coord, rowg, colg, rowscat, nacc):
    h = _lin(p["emb_in"], x)
    for gp in p["gcls"]:
        h, coord = _egcl_big(gp, h, coord, rowg, colg, rowscat, nacc)
    return _lin(p["emb_out"], h), coord


def _egcl_small(p, h, coord, row, col):
    coord_diff = coord[row] - coord[col]
    radial = jnp.sum(coord_diff ** 2, axis=1, keepdims=True)
    ef = jnp.concatenate([h[row], h[col], radial], axis=1)
    ef = _silu(_lin(p["edge_mlp0"], ef))
    ef = _silu(_lin(p["edge_mlp1"], ef))
    att = jax.nn.sigmoid(_lin(p["att_mlp"], ef))
    ef = ef * att
    cm = _silu(_lin(p["coord_mlp0"], ef))
    cm = jnp.tanh(_lin(p["coord_mlp1"], cm))
    coord = coord + _seg_mean(coord_diff * cm, row, coord.shape[0])
    agg = _seg_sum(ef, row, h.shape[0])
    out = _silu(_lin(p["node_mlp0"], jnp.concatenate([h, agg], axis=1)))
    out = _lin(p["node_mlp1"], out)
    return h + out, coord


def _egnn_small(p, h, x, row, col):
    h = _lin(p["emb_in"], h)
    for gp in p["gcls"]:
        h, x = _egcl_small(gp, h, x, row, col)
    return _lin(p["emb_out"], h), x


def kernel(x_res, x_emb_seq, x_pos, params, edge_index, x_batch):
    n = x_res.shape[0]
    e = edge_index.shape[1]
    n_graphs = x_emb_seq.shape[0]
    row = edge_index[0]
    col = edge_index[1]

    epad = ((e + _BE - 1) // _BE) * _BE
    nacc = n + 16
    rowg = jnp.pad(row, (0, epad - e))
    colg = jnp.pad(col, (0, epad - e))
    rowscat = jnp.pad(row, (0, epad - e), constant_values=n)

    r2, c2 = np.triu_indices(n_graphs, k=1)
    row2 = jnp.asarray(r2, dtype=jnp.int32)
    col2 = jnp.asarray(c2, dtype=jnp.int32)

    out_res, pos_res = _egnn_big(params["egnn1"], x_res, x_pos,
                                 rowg, colg, rowscat, nacc)
    out_res2, pos_res2 = _egnn_big(params["egnn2"], out_res, pos_res,
                                   rowg, colg, rowscat, nacc)
    seq_pos = _seg_mean(x_pos, x_batch, n_graphs)
    out_seq, _ = _egnn_small(params["egnn3"], x_emb_seq, seq_pos, row2, col2)
    out_res4, _ = _egnn_big(params["egnn4"], out_res2, pos_res2,
                            rowg, colg, rowscat, nacc)

    o1 = jax.nn.relu(_bn(params["bnrelu1"], _seg_mean(out_res, x_batch, n_graphs)))
    o2 = jax.nn.relu(_bn(params["bnrelu2"], _seg_mean(out_res2, x_batch, n_graphs)))
    oseq = jax.nn.relu(_bn(params["bnrelu2"], out_seq))
    o4 = jax.nn.relu(_bn(params["bnrelu3"], _seg_mean(out_res4, x_batch, n_graphs)))
    feat = jnp.concatenate([o1, oseq, o2, o4], axis=1)
    x = _bn(params["fc1"]["bn"], _lin(params["fc1"]["lin"], feat))
    x = _lin(params["final"], x)
    return jax.nn.sigmoid(x)
